# baseline (speedup 1.0000x reference)
import functools

import jax
import jax.numpy as jnp
from jax import lax
from jax.experimental import pallas as pl
from jax.experimental.pallas import tpu as pltpu
from jax.experimental.pallas import tpu_sc as plsc

VOCAB_SIZE = 1000000
TAG_VOCAB = 1000
REL_VOCAB = 1000
WORD_DIM = 64
TAG_DIM = 32
REL_DIM = 32
B = 4096
L = 200
N = B * L

NC = 2
NS = 16
NW = NC * NS
BB = B // 128
LO = L // 8
NU_TOT = L * BB
NU = NU_TOT // NW
RC = 2
NCH = NU // RC

_mesh = plsc.VectorSubcoreMesh(
    core_axis_name="c", subcore_axis_name="s", num_cores=NC, num_subcores=NS
)

_params = pltpu.CompilerParams(
    needs_layout_passes=False, use_tc_tiling_on_sc=False,
    disable_bounds_checks=True,
)


def _run_phase(wid, idx_hbm, tab_hbm, out_hbm, idx_v, g_v, tv,
               si, sg, so, d_dim, fix, rc=RC):
    n_dt = d_dim // 8
    nch = NU // rc
    r0 = wid * NU

    def chunk_l_bt(c):
        r = r0 + rc * c
        lo = r // 256
        bt = (r // 8) % 32
        li = r % 8
        return lo * 8 + li, bt

    def fire_idx(b, c):
        pltpu.async_copy(idx_hbm.at[pl.ds(r0 + rc * c, rc)], idx_v.at[b], si[b])

    def wait_idx(b):
        pltpu.make_async_copy(idx_hbm.at[pl.ds(0, rc)], idx_v.at[b], si[b]).wait()

    def fire_g(b):
        for j in range(rc):
            pltpu.async_copy(
                tab_hbm.at[idx_v.at[b].at[j]],
                g_v.at[b].at[pl.ds(j * 128, 128)],
                sg[b],
            )

    def wait_g(b):
        for j in range(rc):
            pltpu.make_async_copy(
                tab_hbm.at[idx_v.at[b].at[j]],
                g_v.at[b].at[pl.ds(j * 128, 128)],
                sg[b],
            ).wait()

    def fire_w(b, c):
        l, bt = chunk_l_bt(c)
        for li in range(rc):
            for dt in range(n_dt):
                pltpu.async_copy(
                    tv.at[b, pl.ds(li * d_dim + dt * 8, 8), pl.ds(0, 128)],
                    out_hbm.at[l + li, dt, bt],
                    so[b],
                )

    def wait_w(b):
        for li in range(rc):
            for dt in range(n_dt):
                pltpu.make_async_copy(
                    tv.at[b, pl.ds(li * d_dim + dt * 8, 8), pl.ds(0, 128)],
                    out_hbm.at[0, dt, 0],
                    so[b],
                ).wait()

    def zero_fix(b):
        def bg_body(bg, _):
            rowi = bg * 16 + lax.iota(jnp.int32, 16)
            iv = idx_v[b, bg // 8, pl.ds((bg % 8) * 16, 16)]
            mask = iv == 0
            nz = plsc.all_reduce_population_count(mask)

            @pl.when(nz[0] > 0)
            def _():
                zz = jnp.zeros((16,), jnp.float32)
                for col in range(d_dim):
                    plsc.store_scatter(
                        g_v.at[b],
                        [rowi, jnp.full((16,), col, jnp.int32)],
                        zz, mask=mask,
                    )
            return 0

        lax.fori_loop(0, rc * 8, bg_body, 0)

    def transpose(b):
        for li in range(rc):
            trow = [
                li * d_dim + d0 + lax.iota(jnp.int32, 16)
                for d0 in range(0, d_dim, 16)
            ]

            @plsc.parallel_loop(0, 128, unroll=4)
            def _row(r):
                row = li * 128 + r
                rb = jnp.full((16,), 0, jnp.int32) + r
                for d0i in range(d_dim // 16):
                    v = g_v[b, row, pl.ds(d0i * 16, 16)]
                    plsc.store_scatter(tv.at[b], [trow[d0i], rb], v)

    pltpu.sync_copy(idx_hbm.at[pl.ds(r0, rc)], idx_v.at[0])
    fire_g(0)
    fire_idx(1, 1)

    def body(k, _):
        c0 = 2 * k
        c1 = c0 + 1
        wait_g(0)
        wait_idx(1)
        fire_g(1)

        @pl.when(k > 0)
        def _():
            wait_w(0)

        if fix:
            zero_fix(0)
        transpose(0)
        fire_w(0, c0)

        @pl.when(c0 + 2 < nch)
        def _():
            fire_idx(0, c0 + 2)

        wait_g(1)

        @pl.when(c1 + 1 < nch)
        def _():
            wait_idx(0)
            fire_g(0)

        @pl.when(k > 0)
        def _():
            wait_w(1)

        if fix:
            zero_fix(1)
        transpose(1)
        fire_w(1, c1)

        @pl.when(c1 + 2 < nch)
        def _():
            fire_idx(1, c1 + 2)

        return 0

    lax.fori_loop(0, nch // 2, body, 0)
    wait_w(0)
    wait_w(1)


@functools.partial(
    pl.kernel,
    out_type=jax.ShapeDtypeStruct((L, WORD_DIM // 8, BB, 8, 128), jnp.float32),
    mesh=_mesh,
    scratch_types=(
        pltpu.VMEM((2, RC, 128), jnp.int32),
        pltpu.VMEM((2, RC * 128, WORD_DIM), jnp.float32),
        pltpu.VMEM((2, RC * WORD_DIM, 129), jnp.float32),
        pltpu.SemaphoreType.DMA,
        pltpu.SemaphoreType.DMA,
        pltpu.SemaphoreType.DMA,
        pltpu.SemaphoreType.DMA,
        pltpu.SemaphoreType.DMA,
        pltpu.SemaphoreType.DMA,
    ),
    compiler_params=_params,
)
def _emb_word(sent_hbm, wtab_hbm, wout_hbm, widx, gw, tv,
              si0, si1, sg0, sg1, so0, so1):
    wid = lax.axis_index("s") * NC + lax.axis_index("c")
    _run_phase(wid, sent_hbm, wtab_hbm, wout_hbm, widx, gw, tv,
               (si0, si1), (sg0, sg1), (so0, so1), WORD_DIM, fix=True)


@functools.partial(
    pl.kernel,
    out_type=(
        jax.ShapeDtypeStruct((L, TAG_DIM // 8, BB, 8, 128), jnp.float32),
        jax.ShapeDtypeStruct((L, REL_DIM // 8, BB, 8, 128), jnp.float32),
    ),
    mesh=_mesh,
    scratch_types=(
        pltpu.VMEM((2, 4, 128), jnp.int32),
        pltpu.VMEM((2, 4 * 128, TAG_DIM), jnp.float32),
        pltpu.VMEM((2, 4 * TAG_DIM, 129), jnp.float32),
        pltpu.SemaphoreType.DMA,
        pltpu.SemaphoreType.DMA,
        pltpu.SemaphoreType.DMA,
        pltpu.SemaphoreType.DMA,
        pltpu.SemaphoreType.DMA,
        pltpu.SemaphoreType.DMA,
    ),
    compiler_params=_params,
)
def _emb_small(tag_hbm, rel_hbm, ttab_hbm, rtab_hbm, tout_hbm, rout_hbm,
               sidx, gs, tv,
               si0, si1, sg0, sg1, so0, so1):
    wid = lax.axis_index("s") * NC + lax.axis_index("c")
    _run_phase(wid, tag_hbm, ttab_hbm, tout_hbm, sidx, gs, tv,
               (si0, si1), (sg0, sg1), (so0, so1), TAG_DIM, fix=False, rc=4)
    _run_phase(wid, rel_hbm, rtab_hbm, rout_hbm, sidx, gs, tv,
               (si0, si1), (sg0, sg1), (so0, so1), REL_DIM, fix=False, rc=4)


def _phys_view(x):
    return (
        x.reshape(BB, 128, LO, 8).transpose(2, 0, 3, 1).reshape(NU_TOT, 128)
        .astype(jnp.int32)
    )


def _logical_out(a, d):
    return a.transpose(2, 4, 0, 1, 3).reshape(B, L, d)[:, None]


def kernel(sent_inputs, tag_inputs, rel_inputs, word_table, tag_table, rel_table):
    tout, rout = _emb_small(
        _phys_view(tag_inputs), _phys_view(rel_inputs), tag_table, rel_table
    )
    wout = _emb_word(_phys_view(sent_inputs), word_table)
    return (
        _logical_out(wout, WORD_DIM),
        _logical_out(tout, TAG_DIM),
        _logical_out(rout, REL_DIM),
    )


# speedup vs baseline: 1.0654x; 1.0654x over previous
import functools

import jax
import jax.numpy as jnp
from jax import lax
from jax.experimental import pallas as pl
from jax.experimental.pallas import tpu as pltpu
from jax.experimental.pallas import tpu_sc as plsc

VOCAB_SIZE = 1000000
TAG_VOCAB = 1000
REL_VOCAB = 1000
WORD_DIM = 64
TAG_DIM = 32
REL_DIM = 32
B = 4096
L = 200
N = B * L

NC = 2
NS = 16
NW = NC * NS
BB = B // 128
LO = L // 8
NU_TOT = L * BB
NU = NU_TOT // NW
RC = 2
NCH = NU // RC

_mesh = plsc.VectorSubcoreMesh(
    core_axis_name="c", subcore_axis_name="s", num_cores=NC, num_subcores=NS
)

_params = pltpu.CompilerParams(
    needs_layout_passes=False, use_tc_tiling_on_sc=False,
    disable_bounds_checks=True,
)


def _run_phase(wid, idx_hbm, tab_hbm, out_hbm, idx_v, g_v, tv,
               si, sg, so, d_dim, fix):
    n_dt = d_dim // 8
    r0 = wid * NU

    def chunk_l_bt(c):
        r = r0 + RC * c
        lo = r // 256
        bt = (r // 8) % 32
        li = r % 8
        return lo * 8 + li, bt

    def fire_idx(b, c):
        pltpu.async_copy(idx_hbm.at[pl.ds(r0 + RC * c, RC)], idx_v.at[b], si[b])

    def wait_idx(b):
        pltpu.make_async_copy(idx_hbm.at[pl.ds(0, RC)], idx_v.at[b], si[b]).wait()

    def fire_g(b):
        for j in range(RC):
            pltpu.async_copy(
                tab_hbm.at[idx_v.at[b].at[j]],
                g_v.at[b].at[pl.ds(j * 128, 128)],
                sg[b],
            )

    def wait_g(b):
        for j in range(RC):
            pltpu.make_async_copy(
                tab_hbm.at[idx_v.at[b].at[j]],
                g_v.at[b].at[pl.ds(j * 128, 128)],
                sg[b],
            ).wait()

    def fire_w(b, c):
        l, bt = chunk_l_bt(c)
        for li in range(RC):
            for dt in range(n_dt):
                pltpu.async_copy(
                    tv.at[b, pl.ds(li * d_dim + dt * 8, 8), pl.ds(0, 128)],
                    out_hbm.at[l + li, dt, bt],
                    so[b],
                )

    def wait_w(b):
        for li in range(RC):
            for dt in range(n_dt):
                pltpu.make_async_copy(
                    tv.at[b, pl.ds(li * d_dim + dt * 8, 8), pl.ds(0, 128)],
                    out_hbm.at[0, dt, 0],
                    so[b],
                ).wait()

    def zero_fix(b):
        def bg_body(bg, _):
            rowi = bg * 16 + lax.iota(jnp.int32, 16)
            iv = idx_v[b, bg // 8, pl.ds((bg % 8) * 16, 16)]
            mask = iv == 0
            nz = plsc.all_reduce_population_count(mask)

            @pl.when(nz[0] > 0)
            def _():
                zz = jnp.zeros((16,), jnp.float32)
                for col in range(d_dim):
                    plsc.store_scatter(
                        g_v.at[b],
                        [rowi, jnp.full((16,), col, jnp.int32)],
                        zz, mask=mask,
                    )
            return 0

        lax.fori_loop(0, RC * 8, bg_body, 0)

    def transpose(b):
        for li in range(RC):
            trow = [
                li * d_dim + d0 + lax.iota(jnp.int32, 16)
                for d0 in range(0, d_dim, 16)
            ]

            @plsc.parallel_loop(0, 128, unroll=4)
            def _row(r):
                row = li * 128 + r
                rb = jnp.full((16,), 0, jnp.int32) + r
                for d0i in range(d_dim // 16):
                    v = g_v[b, row, pl.ds(d0i * 16, 16)]
                    plsc.store_scatter(tv.at[b], [trow[d0i], rb], v)

    pltpu.sync_copy(idx_hbm.at[pl.ds(r0, RC)], idx_v.at[0])
    fire_g(0)
    fire_idx(1, 1)

    def body(k, _):
        c0 = 2 * k
        c1 = c0 + 1
        wait_g(0)
        wait_idx(1)
        fire_g(1)

        @pl.when(k > 0)
        def _():
            wait_w(0)

        if fix:
            zero_fix(0)
        transpose(0)
        fire_w(0, c0)

        @pl.when(c0 + 2 < NCH)
        def _():
            fire_idx(0, c0 + 2)

        wait_g(1)

        @pl.when(c1 + 1 < NCH)
        def _():
            wait_idx(0)
            fire_g(0)

        @pl.when(k > 0)
        def _():
            wait_w(1)

        if fix:
            zero_fix(1)
        transpose(1)
        fire_w(1, c1)

        @pl.when(c1 + 2 < NCH)
        def _():
            fire_idx(1, c1 + 2)

        return 0

    lax.fori_loop(0, NCH // 2, body, 0)
    wait_w(0)
    wait_w(1)


@functools.partial(
    pl.kernel,
    out_type=jax.ShapeDtypeStruct((L, WORD_DIM // 8, BB, 8, 128), jnp.float32),
    mesh=_mesh,
    scratch_types=(
        pltpu.VMEM((2, RC, 128), jnp.int32),
        pltpu.VMEM((2, RC * 128, WORD_DIM), jnp.float32),
        pltpu.VMEM((2, RC * WORD_DIM, 129), jnp.float32),
        pltpu.SemaphoreType.DMA,
        pltpu.SemaphoreType.DMA,
        pltpu.SemaphoreType.DMA,
        pltpu.SemaphoreType.DMA,
        pltpu.SemaphoreType.DMA,
        pltpu.SemaphoreType.DMA,
    ),
    compiler_params=_params,
)
def _emb_word(sent_hbm, wtab_hbm, wout_hbm, widx, gw, tv,
              si0, si1, sg0, sg1, so0, so1):
    wid = lax.axis_index("s") * NC + lax.axis_index("c")
    _run_phase(wid, sent_hbm, wtab_hbm, wout_hbm, widx, gw, tv,
               (si0, si1), (sg0, sg1), (so0, so1), WORD_DIM, fix=True)


@functools.partial(
    pl.kernel,
    out_type=(
        jax.ShapeDtypeStruct((L, TAG_DIM // 8, BB, 8, 128), jnp.float32),
        jax.ShapeDtypeStruct((L, REL_DIM // 8, BB, 8, 128), jnp.float32),
    ),
    mesh=_mesh,
    scratch_types=(
        pltpu.VMEM((2, RC, 128), jnp.int32),
        pltpu.VMEM((2, RC * 128, TAG_DIM), jnp.float32),
        pltpu.VMEM((2, RC * TAG_DIM, 129), jnp.float32),
        pltpu.SemaphoreType.DMA,
        pltpu.SemaphoreType.DMA,
        pltpu.SemaphoreType.DMA,
        pltpu.SemaphoreType.DMA,
        pltpu.SemaphoreType.DMA,
        pltpu.SemaphoreType.DMA,
    ),
    compiler_params=_params,
)
def _emb_small(tag_hbm, rel_hbm, ttab_hbm, rtab_hbm, tout_hbm, rout_hbm,
               sidx, gs, tv,
               si0, si1, sg0, sg1, so0, so1):
    wid = lax.axis_index("s") * NC + lax.axis_index("c")
    _run_phase(wid, tag_hbm, ttab_hbm, tout_hbm, sidx, gs, tv,
               (si0, si1), (sg0, sg1), (so0, so1), TAG_DIM, fix=False)
    _run_phase(wid, rel_hbm, rtab_hbm, rout_hbm, sidx, gs, tv,
               (si0, si1), (sg0, sg1), (so0, so1), REL_DIM, fix=False)


def _phys_view(x):
    return (
        x.reshape(BB, 128, LO, 8).transpose(2, 0, 3, 1).reshape(NU_TOT, 128)
        .astype(jnp.int32)
    )


def _logical_out(a, d):
    return a.transpose(2, 4, 0, 1, 3).reshape(B, L, d)[:, None]


def kernel(sent_inputs, tag_inputs, rel_inputs, word_table, tag_table, rel_table):
    tout, rout = _emb_small(
        _phys_view(tag_inputs), _phys_view(rel_inputs), tag_table, rel_table
    )
    wout = _emb_word(_phys_view(sent_inputs), word_table)
    return (
        _logical_out(wout, WORD_DIM),
        _logical_out(tout, TAG_DIM),
        _logical_out(rout, REL_DIM),
    )
